# baseline (speedup 1.0000x reference)
import jax
import jax.numpy as jnp
from jax import lax
from jax.experimental import pallas as pl
from jax.experimental.pallas import tpu as pltpu
from jax.experimental.pallas import tpu_sc as plsc

L = 2048
H = 16
T = 2 * L - 1
SW = 3968


def _body(s_hbm, out_hbm, sem):
    nc = 2
    c = lax.axis_index("c")
    s = lax.axis_index("s")
    wid = s * nc + c
    h = wid // nc
    half = wid % nc
    bi0 = half * (L // 2 // 8)

    def fire(b):
        bi = bi0 + b
        o0 = (L - 8) - 8 * bi
        ybase = (o0 // 128) * 128
        u16 = (o0 - ybase) // 8
        pltpu.async_copy(
            s_hbm.at[h, u16, :, pl.ds(pl.multiple_of(ybase, 128), L)],
            out_hbm.at[0, h, pl.ds(pl.multiple_of(8 * bi, 8), 8), :],
            sem,
        )

    def drain():
        pltpu.make_async_copy(
            s_hbm.at[0, 0, :, pl.ds(0, L)],
            out_hbm.at[0, 0, pl.ds(0, 8), :],
            sem,
        ).wait()

    nb = L // 2 // 8
    fire(0)

    def emit(b, _):
        fire(b + 1)
        drain()
        return _

    lax.fori_loop(0, nb - 1, emit, None)
    drain()


@jax.jit
def _run(rel_bias):
    rcp = jnp.pad(rel_bias[::-1], ((0, 129), (0, 0)))
    rows = []
    for u16 in range(16):
        for r in range(8):
            d = 8 * u16 + 7 - r
            rows.append(rcp[d : d + SW])
    stacked = jnp.stack(rows, 0)
    s = jnp.transpose(stacked, (2, 0, 1)).reshape(H, 16, 8, SW)
    k = pl.kernel(
        _body,
        mesh=plsc.VectorSubcoreMesh(core_axis_name="c", subcore_axis_name="s"),
        out_type=jax.ShapeDtypeStruct((1, H, L, L), jnp.float32),
        scratch_types=[pltpu.SemaphoreType.DMA],
    )
    return k(s)


def kernel(rel_bias):
    return _run(rel_bias)


# speedup vs baseline: 4.5446x; 4.5446x over previous
import jax
import jax.numpy as jnp
from jax import lax
from jax.experimental import pallas as pl
from jax.experimental.pallas import tpu as pltpu
from jax.experimental.pallas import tpu_sc as plsc

L = 2048
H = 16
T = 2 * L - 1
SW = 3968
PW = 2944
NG = 16
GM = 8


def _body(s_hbm, out_hbm, p0_v, p1_v, sem_i0, sem_i1, sem_o):
    nc = 2
    c = lax.axis_index("c")
    s = lax.axis_index("s")
    wid = s * nc + c
    h = wid // nc
    half = wid % nc
    bi0 = half * (L // 2 // 8)
    ylo = (1 - half) * 1024

    planes = (p0_v, p1_v)
    sems_i = (sem_i0, sem_i1)

    def load_plane(g, p):
        g = lax.rem(g, NG)
        pltpu.async_copy(
            s_hbm.at[h, NG - 1 - g, :, pl.ds(pl.multiple_of(ylo, 128), PW)],
            planes[p],
            sems_i[p],
        )

    def fire_group(g, p):
        for m in range(GM):
            bi = bi0 + g + NG * m
            pltpu.async_copy(
                planes[p].at[:, pl.ds(896 - 128 * m, L)],
                out_hbm.at[0, h, pl.ds(pl.multiple_of(8 * bi, 8), 8), :],
                sem_o,
            )

    def drain_group(p):
        for _m in range(GM):
            pltpu.make_async_copy(
                planes[p].at[:, pl.ds(0, L)],
                out_hbm.at[0, 0, pl.ds(0, 8), :],
                sem_o,
            ).wait()

    load_plane(0, 0)
    load_plane(1, 1)

    def step(gg, _):
        for p in range(2):
            g = 2 * gg + p
            pltpu.make_async_copy(s_hbm.at[0, 0, :, pl.ds(0, PW)],
                                  planes[p], sems_i[p]).wait()
            fire_group(g, p)
            drain_group(p)
            load_plane(g + 2, p)
        return _

    lax.fori_loop(0, NG // 2, step, None)
    for p in range(2):
        pltpu.make_async_copy(s_hbm.at[0, 0, :, pl.ds(0, PW)],
                              planes[p], sems_i[p]).wait()


@jax.jit
def _run(rel_bias):
    rcp = jnp.pad(rel_bias[::-1], ((0, 129), (0, 0)))
    rows = []
    for u16 in range(NG):
        for r in range(8):
            d = 8 * u16 + 7 - r
            rows.append(rcp[d : d + SW])
    stacked = jnp.stack(rows, 0)
    s = jnp.transpose(stacked, (2, 0, 1)).reshape(H, NG, 8, SW)
    k = pl.kernel(
        _body,
        mesh=plsc.VectorSubcoreMesh(core_axis_name="c", subcore_axis_name="s"),
        out_type=jax.ShapeDtypeStruct((1, H, L, L), jnp.float32),
        scratch_types=[
            pltpu.VMEM((8, PW), jnp.float32),
            pltpu.VMEM((8, PW), jnp.float32),
            pltpu.SemaphoreType.DMA,
            pltpu.SemaphoreType.DMA,
            pltpu.SemaphoreType.DMA,
        ],
    )
    return k(s)


def kernel(rel_bias):
    return _run(rel_bias)
